# Initial kernel scaffold; baseline (speedup 1.0000x reference)
import jax
import jax.numpy as jnp
from jax.experimental import pallas as pl


def kernel(likelihood, local_cellxgene_ix, n_cells, n_genes):
    raise NotImplementedError("write your pallas kernel here")


# speedup vs baseline: 17.6373x; 17.6373x over previous
import functools

import jax
import jax.numpy as jnp
from jax import lax
from jax.experimental import pallas as pl
from jax.experimental.pallas import tpu as pltpu
from jax.experimental.pallas import tpu_sc as plsc

NFRAG = 3200000
LANE = 128
TOT_ROWS = NFRAG // LANE
NC = 2
NS = 16
NW = NC * NS
GROUPS = TOT_ROWS // 8
GBASE = GROUPS // NW
GEXTRA = GROUPS - NW * GBASE
NSEG = 1000000
ACC_PAD = 1000448
ZCHUNK = 15632
STAGE_ROWS = 128
FULL_STAGES = (GBASE * 8) // STAGE_ROWS
TAIL_ROWS = GBASE * 8 - FULL_STAGES * STAGE_ROWS
TAIL_ROWS_X = TAIL_ROWS + 8
WB_SLICE = ACC_PAD // NS
WB_BUF = 16384


def _sc_body(idx_hbm, val_hbm, out_hbm, acc, zbuf, idxv, valv, wbuf):
    c = lax.axis_index("c")
    s = lax.axis_index("s")
    wid = s * NC + c

    def _z(i, carry):
        zbuf[pl.ds(i * 16, 16)] = jnp.zeros((16,), jnp.float32)
        return carry

    lax.fori_loop(0, ZCHUNK // 16, _z, 0)
    for k in range(4):
        pltpu.sync_copy(zbuf, acc.at[pl.ds(s * (4 * ZCHUNK) + k * ZCHUNK, ZCHUNK)])
    plsc.subcore_barrier()

    start_row = (wid * GBASE + jnp.minimum(wid, GEXTRA)) * 8

    def _do_rows(row0, nrows):
        pltpu.sync_copy(idx_hbm.at[pl.ds(row0, nrows)], idxv.at[pl.ds(0, nrows)])
        pltpu.sync_copy(val_hbm.at[pl.ds(row0, nrows)], valv.at[pl.ds(0, nrows)])

        def _sc(j, carry):
            pltpu.sync_copy(valv.at[j], acc.at[idxv.at[j]], add=True)
            return carry

        lax.fori_loop(0, nrows, _sc, 0)

    def _stage(t, carry):
        _do_rows(start_row + t * STAGE_ROWS, STAGE_ROWS)
        return carry

    lax.fori_loop(0, FULL_STAGES, _stage, 0)

    @pl.when(wid < GEXTRA)
    def _():
        _do_rows(start_row + FULL_STAGES * STAGE_ROWS, TAIL_ROWS_X)

    @pl.when(wid >= GEXTRA)
    def _():
        _do_rows(start_row + FULL_STAGES * STAGE_ROWS, TAIL_ROWS)

    plsc.subcore_barrier()

    def _wb(base, sizes):
        off = 0
        for sz in sizes:
            pltpu.sync_copy(acc.at[pl.ds(base + off, sz)], wbuf.at[pl.ds(0, sz)])
            pltpu.sync_copy(
                wbuf.at[pl.ds(0, sz)],
                out_hbm.at[pl.ds(c * NSEG + base + off, sz)],
            )
            off += sz

    @pl.when(s < NS - 1)
    def _():
        _wb(s * WB_SLICE, [WB_BUF, WB_BUF, WB_BUF, WB_SLICE - 3 * WB_BUF])

    @pl.when(s == NS - 1)
    def _():
        last = NSEG - (NS - 1) * WB_SLICE
        _wb(s * WB_SLICE, [WB_BUF, WB_BUF, WB_BUF, last - 3 * WB_BUF])


@functools.partial(
    pl.kernel,
    out_type=jax.ShapeDtypeStruct((NC * NSEG,), jnp.float32),
    mesh=plsc.VectorSubcoreMesh(core_axis_name="c", subcore_axis_name="s"),
    scratch_types=[
        pltpu.VMEM_SHARED((ACC_PAD,), jnp.float32),
        pltpu.VMEM((ZCHUNK,), jnp.float32),
        pltpu.VMEM((STAGE_ROWS, LANE), jnp.int32),
        pltpu.VMEM((STAGE_ROWS, LANE), jnp.float32),
        pltpu.VMEM((WB_BUF,), jnp.float32),
    ],
)
def _sc_segment_sum(idx_hbm, val_hbm, out_hbm, acc, zbuf, idxv, valv, wbuf):
    _sc_body(idx_hbm, val_hbm, out_hbm, acc, zbuf, idxv, valv, wbuf)


def _combine_body(p_ref, o_ref):
    o_ref[...] = p_ref[0] + p_ref[1]


def kernel(likelihood, local_cellxgene_ix, n_cells, n_genes):
    idx2d = local_cellxgene_ix.astype(jnp.int32).reshape(TOT_ROWS, LANE)
    val2d = likelihood.reshape(TOT_ROWS, LANE)
    part = _sc_segment_sum(idx2d, val2d)
    part3 = part.reshape(NC, 1000, 1000)
    out = pl.pallas_call(
        _combine_body,
        out_shape=jax.ShapeDtypeStruct((1000, 1000), jnp.float32),
    )(part3)
    return out


# speedup vs baseline: 26.0295x; 1.4758x over previous
import functools

import jax
import jax.numpy as jnp
from jax import lax
from jax.experimental import pallas as pl
from jax.experimental.pallas import tpu as pltpu
from jax.experimental.pallas import tpu_sc as plsc

NFRAG = 3200000
LANE = 128
TOT_ROWS = NFRAG // LANE
NC = 2
NS = 16
NW = NC * NS
GROUPS = TOT_ROWS // 8
GBASE = GROUPS // NW
GEXTRA = GROUPS - NW * GBASE
NSEG = 1000000
ACC_PAD = 1000448
STAGE_ROWS = 128
STAGE_ELEMS = STAGE_ROWS * LANE
FULL_STAGES = (GBASE * 8) // STAGE_ROWS
TAIL_ELEMS = (GBASE * 8 - FULL_STAGES * STAGE_ROWS) * LANE
TAIL_ELEMS_X = TAIL_ELEMS + 8 * LANE
WB_SLICE = ACC_PAD // NS
WB_BUF = 16384


def _sc_body(idx_hbm, val_hbm, out_hbm, acc, idxv, valv,
             idxt8, valt8, idxt16, valt16):
    c = lax.axis_index("c")
    s = lax.axis_index("s")
    wid = s * NC + c

    def _z(i, carry):
        valv[pl.ds(i * 16, 16)] = jnp.zeros((16,), jnp.float32)
        return carry

    lax.fori_loop(0, STAGE_ELEMS // 16, _z, 0)
    base0 = s * WB_SLICE
    off = 0
    for sz in (WB_BUF, WB_BUF, WB_BUF, WB_SLICE - 3 * WB_BUF):
        pltpu.sync_copy(valv.at[pl.ds(0, sz)], acc.at[pl.ds(base0 + off, sz)])
        off += sz
    plsc.subcore_barrier()

    start_elem = (wid * GBASE + jnp.minimum(wid, GEXTRA)) * 8 * LANE

    def _do_block(elem0, ib, vb):
        pltpu.sync_copy(idx_hbm.at[pl.ds(elem0, ib.shape[0])], ib)
        pltpu.sync_copy(val_hbm.at[pl.ds(elem0, vb.shape[0])], vb)
        pltpu.sync_copy(vb, acc.at[ib], add=True)

    def _stage(t, carry):
        _do_block(start_elem + t * STAGE_ELEMS, idxv, valv)
        return carry

    lax.fori_loop(0, FULL_STAGES, _stage, 0)
    tail_elem = start_elem + FULL_STAGES * STAGE_ELEMS

    @pl.when(wid < GEXTRA)
    def _():
        _do_block(tail_elem, idxt16, valt16)

    @pl.when(wid >= GEXTRA)
    def _():
        _do_block(tail_elem, idxt8, valt8)

    plsc.subcore_barrier()

    def _wb(base, sizes):
        off = 0
        for sz in sizes:
            pltpu.sync_copy(acc.at[pl.ds(base + off, sz)], valv.at[pl.ds(0, sz)])
            pltpu.sync_copy(
                valv.at[pl.ds(0, sz)],
                out_hbm.at[pl.ds(c * NSEG + base + off, sz)],
            )
            off += sz

    @pl.when(s < NS - 1)
    def _():
        _wb(s * WB_SLICE, [WB_BUF, WB_BUF, WB_BUF, WB_SLICE - 3 * WB_BUF])

    @pl.when(s == NS - 1)
    def _():
        last = NSEG - (NS - 1) * WB_SLICE
        _wb(s * WB_SLICE, [WB_BUF, WB_BUF, WB_BUF, last - 3 * WB_BUF])


@functools.partial(
    pl.kernel,
    out_type=jax.ShapeDtypeStruct((NC * NSEG,), jnp.float32),
    mesh=plsc.VectorSubcoreMesh(core_axis_name="c", subcore_axis_name="s"),
    scratch_types=[
        pltpu.VMEM_SHARED((ACC_PAD,), jnp.float32),
        pltpu.VMEM((STAGE_ELEMS,), jnp.int32),
        pltpu.VMEM((STAGE_ELEMS,), jnp.float32),
        pltpu.VMEM((TAIL_ELEMS,), jnp.int32),
        pltpu.VMEM((TAIL_ELEMS,), jnp.float32),
        pltpu.VMEM((TAIL_ELEMS_X,), jnp.int32),
        pltpu.VMEM((TAIL_ELEMS_X,), jnp.float32),
    ],
)
def _sc_segment_sum(idx_hbm, val_hbm, out_hbm, acc, idxv, valv,
                    idxt8, valt8, idxt16, valt16):
    _sc_body(idx_hbm, val_hbm, out_hbm, acc, idxv, valv,
             idxt8, valt8, idxt16, valt16)


def _combine_body(p_ref, o_ref):
    o_ref[...] = p_ref[0] + p_ref[1]


def kernel(likelihood, local_cellxgene_ix, n_cells, n_genes):
    idx1d = local_cellxgene_ix.astype(jnp.int32)
    part = _sc_segment_sum(idx1d, likelihood)
    part3 = part.reshape(NC, 1000, 1000)
    out = pl.pallas_call(
        _combine_body,
        out_shape=jax.ShapeDtypeStruct((1000, 1000), jnp.float32),
    )(part3)
    return out


# speedup vs baseline: 31.3101x; 1.2029x over previous
import functools

import jax
import jax.numpy as jnp
from jax import lax
from jax.experimental import pallas as pl
from jax.experimental.pallas import tpu as pltpu
from jax.experimental.pallas import tpu_sc as plsc

NFRAG = 3200000
LANE = 128
TOT_ROWS = NFRAG // LANE
NC = 2
NS = 16
NW = NC * NS
GROUPS = TOT_ROWS // 8
GBASE = GROUPS // NW
GEXTRA = GROUPS - NW * GBASE
NSEG = 1000000
ACC_PAD = 1000448
STAGE_ELEMS = 8192
FULL_STAGES = (GBASE * 8 * LANE) // STAGE_ELEMS
TAIL_ELEMS = GBASE * 8 * LANE - FULL_STAGES * STAGE_ELEMS
TAIL_ELEMS_X = TAIL_ELEMS + 8 * LANE
WB_SLICE = ACC_PAD // NS
ZB = 2048


def _sc_body(idx_hbm, val_hbm, out_hbm, acc, idxv0, valv0, idxv1, valv1,
             idxt8, valt8, idxt16, valt16, zb, sems):
    c = lax.axis_index("c")
    s = lax.axis_index("s")
    wid = s * NC + c
    idxb = (idxv0, idxv1)
    valb = (valv0, valv1)
    start_elem = (wid * GBASE + jnp.minimum(wid, GEXTRA)) * 8 * LANE

    ins = {}

    def _stage_start(t):
        b = t % 2
        e0 = start_elem + t * STAGE_ELEMS
        ins[t] = (
            pltpu.async_copy(idx_hbm.at[pl.ds(e0, STAGE_ELEMS)], idxb[b],
                             sems.at[b]),
            pltpu.async_copy(val_hbm.at[pl.ds(e0, STAGE_ELEMS)], valb[b],
                             sems.at[2 + b]),
        )

    _stage_start(0)
    _stage_start(1)

    def _z(i, carry):
        zb[pl.ds(i * 16, 16)] = jnp.zeros((16,), jnp.float32)
        return carry

    lax.fori_loop(0, ZB // 16, _z, 0)
    base0 = s * WB_SLICE
    zhs = []
    off = 0
    for sz in [ZB] * (WB_SLICE // ZB) + [WB_SLICE - (WB_SLICE // ZB) * ZB]:
        if sz:
            zhs.append(pltpu.async_copy(
                zb.at[pl.ds(0, sz)], acc.at[pl.ds(base0 + off, sz)],
                sems.at[4]))
        off += sz
    for h in zhs:
        h.wait()
    plsc.subcore_barrier()

    scs = {}
    for t in range(FULL_STAGES):
        b = t % 2
        for h in ins[t]:
            h.wait()
        scs[t] = pltpu.async_copy(valb[b], acc.at[idxb[b]], sems.at[5 + b],
                                  add=True)
        if t >= 1:
            scs[t - 1].wait()
        if t >= 1 and t + 1 < FULL_STAGES:
            _stage_start(t + 1)
    scs[FULL_STAGES - 1].wait()

    tail_elem = start_elem + FULL_STAGES * STAGE_ELEMS

    def _do_tail(ib, vb):
        pltpu.sync_copy(idx_hbm.at[pl.ds(tail_elem, ib.shape[0])], ib)
        pltpu.sync_copy(val_hbm.at[pl.ds(tail_elem, vb.shape[0])], vb)
        pltpu.sync_copy(vb, acc.at[ib], add=True)

    @pl.when(wid < GEXTRA)
    def _():
        _do_tail(idxt16, valt16)

    @pl.when(wid >= GEXTRA)
    def _():
        _do_tail(idxt8, valt8)

    plsc.subcore_barrier()

    last = NSEG - (NS - 1) * WB_SLICE

    def _wb(total):
        nfull = total // STAGE_ELEMS
        sizes = [STAGE_ELEMS] * nfull + [total - nfull * STAGE_ELEMS]
        outh = [None, None]
        off = 0
        for k, sz in enumerate(sizes):
            if sz == 0:
                continue
            b = k % 2
            if outh[b] is not None:
                outh[b].wait()
            pltpu.sync_copy(acc.at[pl.ds(base0 + off, sz)],
                            valb[b].at[pl.ds(0, sz)])
            outh[b] = pltpu.async_copy(
                valb[b].at[pl.ds(0, sz)],
                out_hbm.at[pl.ds(c * NSEG + base0 + off, sz)],
                sems.at[5 + b])
            off += sz
        for h in outh:
            if h is not None:
                h.wait()

    @pl.when(s < NS - 1)
    def _():
        _wb(WB_SLICE)

    @pl.when(s == NS - 1)
    def _():
        _wb(last)


@functools.partial(
    pl.kernel,
    out_type=jax.ShapeDtypeStruct((NC * NSEG,), jnp.float32),
    mesh=plsc.VectorSubcoreMesh(core_axis_name="c", subcore_axis_name="s"),
    scratch_types=[
        pltpu.VMEM_SHARED((ACC_PAD,), jnp.float32),
        pltpu.VMEM((STAGE_ELEMS,), jnp.int32),
        pltpu.VMEM((STAGE_ELEMS,), jnp.float32),
        pltpu.VMEM((STAGE_ELEMS,), jnp.int32),
        pltpu.VMEM((STAGE_ELEMS,), jnp.float32),
        pltpu.VMEM((TAIL_ELEMS,), jnp.int32),
        pltpu.VMEM((TAIL_ELEMS,), jnp.float32),
        pltpu.VMEM((TAIL_ELEMS_X,), jnp.int32),
        pltpu.VMEM((TAIL_ELEMS_X,), jnp.float32),
        pltpu.VMEM((ZB,), jnp.float32),
        pltpu.SemaphoreType.DMA((7,)),
    ],
)
def _sc_segment_sum(idx_hbm, val_hbm, out_hbm, acc, idxv0, valv0,
                    idxv1, valv1, idxt8, valt8, idxt16, valt16, zb, sems):
    _sc_body(idx_hbm, val_hbm, out_hbm, acc, idxv0, valv0, idxv1, valv1,
             idxt8, valt8, idxt16, valt16, zb, sems)


def _combine_body(p_ref, o_ref):
    o_ref[...] = p_ref[0] + p_ref[1]


def kernel(likelihood, local_cellxgene_ix, n_cells, n_genes):
    idx1d = local_cellxgene_ix.astype(jnp.int32)
    part = _sc_segment_sum(idx1d, likelihood)
    part3 = part.reshape(NC, 1000, 1000)
    out = pl.pallas_call(
        _combine_body,
        out_shape=jax.ShapeDtypeStruct((1000, 1000), jnp.float32),
    )(part3)
    return out
